# baseline (speedup 1.0000x reference)
import functools

import jax
import jax.numpy as jnp
from jax import lax
from jax.experimental import pallas as pl
from jax.experimental.pallas import tpu as pltpu
from jax.experimental.pallas import tpu_sc as plsc

NUM_CORES = 2
NUM_SUBCORES = 16
NUM_WORKERS = NUM_CORES * NUM_SUBCORES
CHUNK = 32
NBUF = 4


def _gather_fn(B, D):
    bpw = B // NUM_WORKERS
    nchunks = bpw // CHUNK
    npairs = nchunks // NBUF
    mesh = plsc.VectorSubcoreMesh(core_axis_name="c", subcore_axis_name="s")

    @functools.partial(
        pl.kernel,
        out_type=jax.ShapeDtypeStruct((B, D), jnp.float32),
        mesh=mesh,
        scratch_types=[
            pltpu.VMEM((bpw,), jnp.int32),
            [pltpu.VMEM((CHUNK, D), jnp.float32) for _ in range(NBUF)],
            [pltpu.SemaphoreType.DMA for _ in range(NBUF)],
            [pltpu.SemaphoreType.DMA for _ in range(NBUF)],
        ],
    )
    def gather_kernel(idx_hbm, table_hbm, out_hbm, idx_v, bufs, gsems, osems):
        wid = lax.axis_index("s") * NUM_CORES + lax.axis_index("c")
        base = wid * bpw
        pltpu.sync_copy(idx_hbm.at[pl.ds(base, bpw)], idx_v)

        def g_copy(k, j):
            return pltpu.make_async_copy(
                table_hbm.at[idx_v.at[pl.ds(k * CHUNK, CHUNK)]],
                bufs[j], gsems[j],
            )

        def w_copy(k, j):
            return pltpu.make_async_copy(
                bufs[j], out_hbm.at[pl.ds(base + k * CHUNK, CHUNK)], osems[j]
            )

        for j in range(NBUF):
            g_copy(j, j).start()

        def step(k, j):
            g_copy(k, j).wait()
            w_copy(k, j).start()
            w_copy(k, j).wait()
            g_copy(k + NBUF, j).start()

        full_rounds = (nchunks - NBUF) // NBUF
        rem = (nchunks - NBUF) % NBUF

        def round_body(p, carry):
            for j in range(NBUF):
                step(p * NBUF + j, j)
            return carry

        lax.fori_loop(0, full_rounds, round_body, 0)

        for t in range(rem):
            k = full_rounds * NBUF + t
            step(k, k % NBUF)
        for t in range(NBUF):
            k = nchunks - NBUF + t
            g_copy(k, k % NBUF).wait()
            w_copy(k, k % NBUF).start()
        for t in range(NBUF):
            k = nchunks - NBUF + t
            w_copy(k, k % NBUF).wait()

    return gather_kernel


@jax.jit
def kernel(indices, table):
    B = indices.size
    V, D = table.shape
    idx_flat = indices.reshape(B).astype(jnp.int32)
    out = _gather_fn(B, D)(idx_flat, table)
    return out.reshape(indices.shape + (D,))


# speedup vs baseline: 1.0022x; 1.0022x over previous
import functools

import jax
import jax.numpy as jnp
from jax import lax
from jax.experimental import pallas as pl
from jax.experimental.pallas import tpu as pltpu
from jax.experimental.pallas import tpu_sc as plsc

NUM_CORES = 2
NUM_SUBCORES = 16
NUM_WORKERS = NUM_CORES * NUM_SUBCORES
CHUNK = 32
NBUF = 6


def _gather_fn(B, D):
    bpw = B // NUM_WORKERS
    nchunks = bpw // CHUNK
    npairs = nchunks // NBUF
    mesh = plsc.VectorSubcoreMesh(core_axis_name="c", subcore_axis_name="s")

    @functools.partial(
        pl.kernel,
        out_type=jax.ShapeDtypeStruct((B, D), jnp.float32),
        mesh=mesh,
        scratch_types=[
            pltpu.VMEM((bpw,), jnp.int32),
            [pltpu.VMEM((CHUNK, D), jnp.float32) for _ in range(NBUF)],
            [pltpu.SemaphoreType.DMA for _ in range(NBUF)],
            [pltpu.SemaphoreType.DMA for _ in range(NBUF)],
            pltpu.SemaphoreType.DMA,
        ],
    )
    def gather_kernel(idx_hbm, table_hbm, out_hbm, idx_v, bufs, gsems, osems,
                      isem):
        wid = lax.axis_index("s") * NUM_CORES + lax.axis_index("c")
        base = wid * bpw
        head = NBUF * CHUNK
        pltpu.sync_copy(idx_hbm.at[pl.ds(base, head)], idx_v.at[pl.ds(0, head)])
        tail = pltpu.make_async_copy(
            idx_hbm.at[pl.ds(base + head, bpw - head)],
            idx_v.at[pl.ds(head, bpw - head)], isem,
        )
        tail.start()

        def g_copy(k, j):
            return pltpu.make_async_copy(
                table_hbm.at[idx_v.at[pl.ds(k * CHUNK, CHUNK)]],
                bufs[j], gsems[j],
            )

        def w_copy(k, j):
            return pltpu.make_async_copy(
                bufs[j], out_hbm.at[pl.ds(base + k * CHUNK, CHUNK)], osems[j]
            )

        for j in range(NBUF):
            g_copy(j, j).start()
        tail.wait()

        def step(k, j):
            g_copy(k, j).wait()
            w_copy(k, j).start()
            w_copy(k, j).wait()
            g_copy(k + NBUF, j).start()

        full_rounds = (nchunks - NBUF) // NBUF
        rem = (nchunks - NBUF) % NBUF

        def round_body(p, carry):
            for j in range(NBUF):
                step(p * NBUF + j, j)
            return carry

        lax.fori_loop(0, full_rounds, round_body, 0)

        for t in range(rem):
            k = full_rounds * NBUF + t
            step(k, k % NBUF)
        for t in range(NBUF):
            k = nchunks - NBUF + t
            g_copy(k, k % NBUF).wait()
            w_copy(k, k % NBUF).start()
        for t in range(NBUF):
            k = nchunks - NBUF + t
            w_copy(k, k % NBUF).wait()

    return gather_kernel


@jax.jit
def kernel(indices, table):
    B = indices.size
    V, D = table.shape
    idx_flat = indices.reshape(B).astype(jnp.int32)
    out = _gather_fn(B, D)(idx_flat, table)
    return out.reshape(indices.shape + (D,))
